# baseline (speedup 1.0000x reference)
import functools

import jax
import jax.numpy as jnp
from jax import lax
from jax.experimental import pallas as pl
from jax.experimental.pallas import tpu as pltpu

f32 = jnp.float32

N = 50000
E = 800000
H = 64

BN = 2000
BE = 8000


def _embed_kernel(x_ref, w_ref, out_ref):
    out_ref[...] = jnp.dot(x_ref[...], w_ref[...], preferred_element_type=f32)


def _embed(x, w, blk):
    n = x.shape[0]
    return pl.pallas_call(
        _embed_kernel,
        grid=(n // blk,),
        in_specs=[
            pl.BlockSpec((blk, x.shape[1]), lambda i: (i, 0)),
            pl.BlockSpec(w.shape, lambda i: (0, 0)),
        ],
        out_specs=pl.BlockSpec((blk, w.shape[1]), lambda i: (i, 0)),
        out_shape=jax.ShapeDtypeStruct((n, w.shape[1]), f32),
    )(x, w)


def _mlp_kernel(h_ref, agg_ref, w1_ref, b1_ref, w2_ref, b2_ref, out_ref):
    h = h_ref[...]
    z = h + agg_ref[...]
    z = jnp.maximum(jnp.dot(z, w1_ref[...], preferred_element_type=f32)
                    + b1_ref[...], 0.0)
    z = jnp.dot(z, w2_ref[...], preferred_element_type=f32) + b2_ref[...]
    out_ref[...] = jnp.maximum(z, 0.0) + h


def _mlp(h, agg, w1, b1, w2, b2):
    return pl.pallas_call(
        _mlp_kernel,
        grid=(N // BN,),
        in_specs=[
            pl.BlockSpec((BN, H), lambda i: (i, 0)),
            pl.BlockSpec((BN, H), lambda i: (i, 0)),
            pl.BlockSpec((H, H), lambda i: (0, 0)),
            pl.BlockSpec((1, H), lambda i: (0, 0)),
            pl.BlockSpec((H, H), lambda i: (0, 0)),
            pl.BlockSpec((1, H), lambda i: (0, 0)),
        ],
        out_specs=pl.BlockSpec((BN, H), lambda i: (i, 0)),
        out_shape=jax.ShapeDtypeStruct((N, H), f32),
    )(h, agg, w1, b1, w2, b2)


def _mlp_readout_kernel(h_ref, agg_ref, w1_ref, b1_ref, w2_ref, b2_ref,
                        wo_ref, bo_ref, out_ref, acc_ref):
    i = pl.program_id(0)
    h = h_ref[...]
    z = h + agg_ref[...]
    z = jnp.maximum(jnp.dot(z, w1_ref[...], preferred_element_type=f32)
                    + b1_ref[...], 0.0)
    z = jnp.dot(z, w2_ref[...], preferred_element_type=f32) + b2_ref[...]
    hn = jnp.maximum(z, 0.0) + h
    part = jnp.sum(hn.reshape(-1, 8, H), axis=0)

    @pl.when(i == 0)
    def _():
        acc_ref[...] = part

    @pl.when(i > 0)
    def _():
        acc_ref[...] = acc_ref[...] + part

    @pl.when(i == pl.num_programs(0) - 1)
    def _():
        tot = jnp.sum(acc_ref[...], axis=0, keepdims=True)
        out_ref[...] = (jnp.sum(tot * wo_ref[...], axis=1, keepdims=True)
                        + bo_ref[...])


def _mlp_readout(h, agg, w1, b1, w2, b2, wo_r, bo_r):
    return pl.pallas_call(
        _mlp_readout_kernel,
        grid=(N // BN,),
        in_specs=[
            pl.BlockSpec((BN, H), lambda i: (i, 0)),
            pl.BlockSpec((BN, H), lambda i: (i, 0)),
            pl.BlockSpec((H, H), lambda i: (0, 0)),
            pl.BlockSpec((1, H), lambda i: (0, 0)),
            pl.BlockSpec((H, H), lambda i: (0, 0)),
            pl.BlockSpec((1, H), lambda i: (0, 0)),
            pl.BlockSpec((1, H), lambda i: (0, 0)),
            pl.BlockSpec((1, 1), lambda i: (0, 0)),
        ],
        out_specs=pl.BlockSpec((1, 1), lambda i: (0, 0)),
        out_shape=jax.ShapeDtypeStruct((1, 1), f32),
        scratch_shapes=[pltpu.VMEM((8, H), f32)],
    )(h, agg, w1, b1, w2, b2, wo_r, bo_r)


def kernel(feat, eweight, edge_index, Wn, We, params, Wo, bo):
    feat_p = jnp.pad(feat, ((0, 0), (0, H - feat.shape[1])))
    Wn_p = jnp.pad(Wn, ((0, H - Wn.shape[0]), (0, 0)))
    ew_p = jnp.pad(eweight, ((0, 0), (0, 8 - eweight.shape[1])))
    We_p = jnp.pad(We, ((0, 8 - We.shape[0]), (0, 0)))

    h = _embed(feat_p, Wn_p, BN)
    e = _embed(ew_p, We_p, BE)
    src = edge_index[0]
    dst = edge_index[1]

    wo_r = Wo.reshape(1, H)
    bo_r = bo.reshape(1, 1)

    out = None
    for li, (W1, b1, W2, b2) in enumerate(params):
        m = jnp.maximum(h[src] + e, 0.0)
        agg = jax.ops.segment_sum(m, dst, num_segments=N)
        b1r = b1.reshape(1, H)
        b2r = b2.reshape(1, H)
        if li < 3:
            h = _mlp(h, agg, W1, b1r, W2, b2r)
        else:
            out = _mlp_readout(h, agg, W1, b1r, W2, b2r, wo_r, bo_r)
    return out


# speedup vs baseline: 1.8320x; 1.8320x over previous
import functools

import jax
import jax.numpy as jnp
from jax import lax
from jax.experimental import pallas as pl
from jax.experimental.pallas import tpu as pltpu
from jax.experimental.pallas import tpu_sc as plsc

f32 = jnp.float32

N = 50000
E = 800000
H = 64
HH = H // 2

BN = 2000
BE = 8000

NSUB = 16
EPT = E // NSUB
K = 80
NCHUNK = EPT // K
N_PAD = 50176
ROWS_PT = N_PAD // NSUB
ZROWS = 784


def _embed2_kernel(x_ref, w_ref, lo_ref, hi_ref):
    z = jnp.dot(x_ref[...], w_ref[...], preferred_element_type=f32)
    lo_ref[...] = z[:, :HH]
    hi_ref[...] = z[:, HH:]


def _embed2(x, w, blk):
    n = x.shape[0]
    return pl.pallas_call(
        _embed2_kernel,
        grid=(n // blk,),
        in_specs=[
            pl.BlockSpec((blk, x.shape[1]), lambda i: (i, 0)),
            pl.BlockSpec(w.shape, lambda i: (0, 0)),
        ],
        out_specs=[
            pl.BlockSpec((blk, HH), lambda i: (i, 0)),
            pl.BlockSpec((blk, HH), lambda i: (i, 0)),
        ],
        out_shape=[
            jax.ShapeDtypeStruct((n, HH), f32),
            jax.ShapeDtypeStruct((n, HH), f32),
        ],
    )(x, w)


def _mlp2_kernel(hlo_ref, hhi_ref, alo_ref, ahi_ref,
                 w1_ref, b1_ref, w2_ref, b2_ref, olo_ref, ohi_ref):
    h = jnp.concatenate([hlo_ref[...], hhi_ref[...]], axis=1)
    z = h + jnp.concatenate([alo_ref[...], ahi_ref[...]], axis=1)
    z = jnp.maximum(jnp.dot(z, w1_ref[...], preferred_element_type=f32)
                    + b1_ref[...], 0.0)
    z = jnp.dot(z, w2_ref[...], preferred_element_type=f32) + b2_ref[...]
    hn = jnp.maximum(z, 0.0) + h
    olo_ref[...] = hn[:, :HH]
    ohi_ref[...] = hn[:, HH:]


def _mlp2(hlo, hhi, alo, ahi, w1, b1, w2, b2):
    return pl.pallas_call(
        _mlp2_kernel,
        grid=(N // BN,),
        in_specs=[
            pl.BlockSpec((BN, HH), lambda i: (i, 0)),
            pl.BlockSpec((BN, HH), lambda i: (i, 0)),
            pl.BlockSpec((BN, HH), lambda i: (i, 0)),
            pl.BlockSpec((BN, HH), lambda i: (i, 0)),
            pl.BlockSpec((H, H), lambda i: (0, 0)),
            pl.BlockSpec((1, H), lambda i: (0, 0)),
            pl.BlockSpec((H, H), lambda i: (0, 0)),
            pl.BlockSpec((1, H), lambda i: (0, 0)),
        ],
        out_specs=[
            pl.BlockSpec((BN, HH), lambda i: (i, 0)),
            pl.BlockSpec((BN, HH), lambda i: (i, 0)),
        ],
        out_shape=[
            jax.ShapeDtypeStruct((N, HH), f32),
            jax.ShapeDtypeStruct((N, HH), f32),
        ],
    )(hlo, hhi, alo, ahi, w1, b1, w2, b2)


def _mlp_readout_kernel(hlo_ref, hhi_ref, alo_ref, ahi_ref,
                        w1_ref, b1_ref, w2_ref, b2_ref,
                        wo_ref, bo_ref, out_ref, acc_ref):
    i = pl.program_id(0)
    h = jnp.concatenate([hlo_ref[...], hhi_ref[...]], axis=1)
    z = h + jnp.concatenate([alo_ref[...], ahi_ref[...]], axis=1)
    z = jnp.maximum(jnp.dot(z, w1_ref[...], preferred_element_type=f32)
                    + b1_ref[...], 0.0)
    z = jnp.dot(z, w2_ref[...], preferred_element_type=f32) + b2_ref[...]
    hn = jnp.maximum(z, 0.0) + h
    part = jnp.sum(hn.reshape(-1, 8, H), axis=0)

    @pl.when(i == 0)
    def _():
        acc_ref[...] = part

    @pl.when(i > 0)
    def _():
        acc_ref[...] = acc_ref[...] + part

    @pl.when(i == pl.num_programs(0) - 1)
    def _():
        tot = jnp.sum(acc_ref[...], axis=0, keepdims=True)
        out_ref[...] = (jnp.sum(tot * wo_ref[...], axis=1, keepdims=True)
                        + bo_ref[...])


def _mlp_readout(hlo, hhi, alo, ahi, w1, b1, w2, b2, wo_r, bo_r):
    return pl.pallas_call(
        _mlp_readout_kernel,
        grid=(N // BN,),
        in_specs=[
            pl.BlockSpec((BN, HH), lambda i: (i, 0)),
            pl.BlockSpec((BN, HH), lambda i: (i, 0)),
            pl.BlockSpec((BN, HH), lambda i: (i, 0)),
            pl.BlockSpec((BN, HH), lambda i: (i, 0)),
            pl.BlockSpec((H, H), lambda i: (0, 0)),
            pl.BlockSpec((1, H), lambda i: (0, 0)),
            pl.BlockSpec((H, H), lambda i: (0, 0)),
            pl.BlockSpec((1, H), lambda i: (0, 0)),
            pl.BlockSpec((1, H), lambda i: (0, 0)),
            pl.BlockSpec((1, 1), lambda i: (0, 0)),
        ],
        out_specs=pl.BlockSpec((1, 1), lambda i: (0, 0)),
        out_shape=jax.ShapeDtypeStruct((1, 1), f32),
        scratch_shapes=[pltpu.VMEM((8, H), f32)],
    )(hlo, hhi, alo, ahi, w1, b1, w2, b2, wo_r, bo_r)


def _edge_half(s, h_ref, e_ref, src_hbm, dst_hbm, out_ref,
               sidx, didx, rows, zbuf, spacc, sem):
    def zrow(i, _):
        zbuf[i, pl.ds(0, 16)] = jnp.zeros((16,), f32)
        zbuf[i, pl.ds(16, 16)] = jnp.zeros((16,), f32)
        return _
    lax.fori_loop(0, ZROWS, zrow, None, unroll=4)
    base = s * ROWS_PT
    for j in range(ROWS_PT // ZROWS):
        pltpu.sync_copy(zbuf, spacc.at[pl.ds(base + j * ZROWS, ZROWS)])
    plsc.subcore_barrier()

    def chunk(i, _):
        eb = s * EPT + i * K
        pltpu.sync_copy(e_ref.at[pl.ds(eb, K)], rows)
        pltpu.sync_copy(src_hbm.at[pl.ds(eb, K)], sidx)
        pltpu.sync_copy(dst_hbm.at[pl.ds(eb, K)], didx)
        pltpu.async_copy(h_ref.at[sidx], rows, sem, add=True).wait()

        def rrow(r, _):
            rows[r, pl.ds(0, 16)] = jnp.maximum(rows[r, pl.ds(0, 16)], 0.0)
            rows[r, pl.ds(16, 16)] = jnp.maximum(rows[r, pl.ds(16, 16)], 0.0)
            return _
        lax.fori_loop(0, K, rrow, None, unroll=8)
        pltpu.sync_copy(rows, spacc.at[didx], add=True)
        return _
    lax.fori_loop(0, NCHUNK, chunk, None)
    plsc.subcore_barrier()
    for j in range(ROWS_PT // ZROWS):
        sl = pl.ds(base + j * ZROWS, ZROWS)
        pltpu.sync_copy(spacc.at[sl], out_ref.at[sl])


def _edge_body(hlo, hhi, elo, ehi, src_hbm, dst_hbm, agg_lo, agg_hi,
               sidx, didx, rows, zbuf, spacc, sem):
    c = lax.axis_index("c")
    s = lax.axis_index("s")

    @pl.when(c == 0)
    def _():
        _edge_half(s, hlo, elo, src_hbm, dst_hbm, agg_lo,
                   sidx, didx, rows, zbuf, spacc, sem)

    @pl.when(c == 1)
    def _():
        _edge_half(s, hhi, ehi, src_hbm, dst_hbm, agg_hi,
                   sidx, didx, rows, zbuf, spacc, sem)


def _edge_call(hlo, hhi, elo, ehi, src, dst):
    mesh = plsc.VectorSubcoreMesh(core_axis_name="c", subcore_axis_name="s")
    fn = pl.kernel(
        _edge_body,
        out_type=(
            jax.ShapeDtypeStruct((N_PAD, HH), f32),
            jax.ShapeDtypeStruct((N_PAD, HH), f32),
        ),
        mesh=mesh,
        scratch_types=[
            pltpu.VMEM((K,), jnp.int32),
            pltpu.VMEM((K,), jnp.int32),
            pltpu.VMEM((K, HH), f32),
            pltpu.VMEM((ZROWS, HH), f32),
            pltpu.VMEM_SHARED((N_PAD, HH), f32),
            pltpu.SemaphoreType.DMA,
        ],
        compiler_params=pltpu.CompilerParams(use_tc_tiling_on_sc=False),
    )
    return fn(hlo, hhi, elo, ehi, src, dst)


def kernel(feat, eweight, edge_index, Wn, We, params, Wo, bo):
    feat_p = jnp.pad(feat, ((0, 0), (0, H - feat.shape[1])))
    Wn_p = jnp.pad(Wn, ((0, H - Wn.shape[0]), (0, 0)))
    ew_p = jnp.pad(eweight, ((0, 0), (0, 8 - eweight.shape[1])))
    We_p = jnp.pad(We, ((0, 8 - We.shape[0]), (0, 0)))

    h_lo, h_hi = _embed2(feat_p, Wn_p, BN)
    e_lo, e_hi = _embed2(ew_p, We_p, BE)
    src = edge_index[0]
    dst = edge_index[1]

    wo_r = Wo.reshape(1, H)
    bo_r = bo.reshape(1, 1)

    out = None
    for li, (W1, b1, W2, b2) in enumerate(params):
        agg_lo, agg_hi = _edge_call(h_lo, h_hi, e_lo, e_hi, src, dst)
        b1r = b1.reshape(1, H)
        b2r = b2.reshape(1, H)
        if li < 3:
            h_lo, h_hi = _mlp2(h_lo, h_hi, agg_lo, agg_hi, W1, b1r, W2, b2r)
        else:
            out = _mlp_readout(h_lo, h_hi, agg_lo, agg_hi,
                               W1, b1r, W2, b2r, wo_r, bo_r)
    return out


# speedup vs baseline: 4.3950x; 2.3990x over previous
import functools

import jax
import jax.numpy as jnp
from jax import lax
from jax.experimental import pallas as pl
from jax.experimental.pallas import tpu as pltpu
from jax.experimental.pallas import tpu_sc as plsc

f32 = jnp.float32

N = 50000
E = 800000
H = 64
HH = H // 2

BN = 2000
BE = 8000

NSUB = 16
K = 128
EPT = 50048
CH_A = 391
CH_B = 385
NBUF = 3
N_PAD = 50176
ROWS_PT = N_PAD // NSUB
ZROWS = 392


def _embed2_kernel(x_ref, w_ref, lo_ref, hi_ref):
    z = jnp.dot(x_ref[...], w_ref[...], preferred_element_type=f32)
    lo_ref[...] = z[:, :HH]
    hi_ref[...] = z[:, HH:]


def _embed2(x, w, blk):
    n = x.shape[0]
    return pl.pallas_call(
        _embed2_kernel,
        grid=(n // blk,),
        in_specs=[
            pl.BlockSpec((blk, x.shape[1]), lambda i: (i, 0)),
            pl.BlockSpec(w.shape, lambda i: (0, 0)),
        ],
        out_specs=[
            pl.BlockSpec((blk, HH), lambda i: (i, 0)),
            pl.BlockSpec((blk, HH), lambda i: (i, 0)),
        ],
        out_shape=[
            jax.ShapeDtypeStruct((n, HH), f32),
            jax.ShapeDtypeStruct((n, HH), f32),
        ],
    )(x, w)


def _mlp2_kernel(hlo_ref, hhi_ref, alo_ref, ahi_ref,
                 w1_ref, b1_ref, w2_ref, b2_ref, olo_ref, ohi_ref):
    h = jnp.concatenate([hlo_ref[...], hhi_ref[...]], axis=1)
    z = h + jnp.concatenate([alo_ref[...], ahi_ref[...]], axis=1)
    z = jnp.maximum(jnp.dot(z, w1_ref[...], preferred_element_type=f32)
                    + b1_ref[...], 0.0)
    z = jnp.dot(z, w2_ref[...], preferred_element_type=f32) + b2_ref[...]
    hn = jnp.maximum(z, 0.0) + h
    olo_ref[...] = hn[:, :HH]
    ohi_ref[...] = hn[:, HH:]


def _mlp2(hlo, hhi, alo, ahi, w1, b1, w2, b2):
    return pl.pallas_call(
        _mlp2_kernel,
        grid=(N // BN,),
        in_specs=[
            pl.BlockSpec((BN, HH), lambda i: (i, 0)),
            pl.BlockSpec((BN, HH), lambda i: (i, 0)),
            pl.BlockSpec((BN, HH), lambda i: (i, 0)),
            pl.BlockSpec((BN, HH), lambda i: (i, 0)),
            pl.BlockSpec((H, H), lambda i: (0, 0)),
            pl.BlockSpec((1, H), lambda i: (0, 0)),
            pl.BlockSpec((H, H), lambda i: (0, 0)),
            pl.BlockSpec((1, H), lambda i: (0, 0)),
        ],
        out_specs=[
            pl.BlockSpec((BN, HH), lambda i: (i, 0)),
            pl.BlockSpec((BN, HH), lambda i: (i, 0)),
        ],
        out_shape=[
            jax.ShapeDtypeStruct((N, HH), f32),
            jax.ShapeDtypeStruct((N, HH), f32),
        ],
    )(hlo, hhi, alo, ahi, w1, b1, w2, b2)


def _mlp_readout_kernel(hlo_ref, hhi_ref, alo_ref, ahi_ref,
                        w1_ref, b1_ref, w2_ref, b2_ref,
                        wo_ref, bo_ref, out_ref, acc_ref):
    i = pl.program_id(0)
    h = jnp.concatenate([hlo_ref[...], hhi_ref[...]], axis=1)
    z = h + jnp.concatenate([alo_ref[...], ahi_ref[...]], axis=1)
    z = jnp.maximum(jnp.dot(z, w1_ref[...], preferred_element_type=f32)
                    + b1_ref[...], 0.0)
    z = jnp.dot(z, w2_ref[...], preferred_element_type=f32) + b2_ref[...]
    hn = jnp.maximum(z, 0.0) + h
    part = jnp.sum(hn.reshape(-1, 8, H), axis=0)

    @pl.when(i == 0)
    def _():
        acc_ref[...] = part

    @pl.when(i > 0)
    def _():
        acc_ref[...] = acc_ref[...] + part

    @pl.when(i == pl.num_programs(0) - 1)
    def _():
        tot = jnp.sum(acc_ref[...], axis=0, keepdims=True)
        out_ref[...] = (jnp.sum(tot * wo_ref[...], axis=1, keepdims=True)
                        + bo_ref[...])


def _mlp_readout(hlo, hhi, alo, ahi, w1, b1, w2, b2, wo_r, bo_r):
    return pl.pallas_call(
        _mlp_readout_kernel,
        grid=(N // BN,),
        in_specs=[
            pl.BlockSpec((BN, HH), lambda i: (i, 0)),
            pl.BlockSpec((BN, HH), lambda i: (i, 0)),
            pl.BlockSpec((BN, HH), lambda i: (i, 0)),
            pl.BlockSpec((BN, HH), lambda i: (i, 0)),
            pl.BlockSpec((H, H), lambda i: (0, 0)),
            pl.BlockSpec((1, H), lambda i: (0, 0)),
            pl.BlockSpec((H, H), lambda i: (0, 0)),
            pl.BlockSpec((1, H), lambda i: (0, 0)),
            pl.BlockSpec((1, H), lambda i: (0, 0)),
            pl.BlockSpec((1, 1), lambda i: (0, 0)),
        ],
        out_specs=pl.BlockSpec((1, 1), lambda i: (0, 0)),
        out_shape=jax.ShapeDtypeStruct((1, 1), f32),
        scratch_shapes=[pltpu.VMEM((8, H), f32)],
    )(hlo, hhi, alo, ahi, w1, b1, w2, b2, wo_r, bo_r)


def _edge_half(s, h_ref, e_ref, ei_ref, out_ref,
               sib, ebuf, zbuf, spacc, sem_l, sem_g, sem_s):
    def zrow(i, _):
        zbuf[i, pl.ds(0, 16)] = jnp.zeros((16,), f32)
        zbuf[i, pl.ds(16, 16)] = jnp.zeros((16,), f32)
        return _
    lax.fori_loop(0, ZROWS, zrow, None, unroll=4)
    base = s * ROWS_PT
    for j in range(ROWS_PT // ZROWS):
        pltpu.sync_copy(zbuf, spacc.at[pl.ds(base + j * ZROWS, ZROWS)])
    plsc.subcore_barrier()

    n = jnp.where(s < NSUB - 1, CH_A, CH_B)
    ebase = s * EPT

    def load_copies(i, b):
        sl = pl.ds(ebase + i * K, K)
        return (pltpu.make_async_copy(e_ref.at[sl], ebuf.at[b], sem_l),
                pltpu.make_async_copy(ei_ref.at[:, sl], sib.at[b], sem_l))

    def start_loads(i, b):
        for d in load_copies(i, b):
            d.start()

    def wait_loads(i, b):
        for d in load_copies(i, b):
            d.wait()

    def gather_desc(i, b):
        return pltpu.make_async_copy(h_ref.at[sib.at[b, 0]], ebuf.at[b],
                                     sem_g)

    def scatter_desc(b):
        return pltpu.make_async_copy(ebuf.at[b], spacc.at[sib.at[b, 1]],
                                     sem_s)

    start_loads(0, 0)
    wait_loads(0, 0)
    gather_desc(0, 0).start(add=True)
    start_loads(1, 1)

    def group(g, _):
        for b in range(NBUF):
            i = g * NBUF + b

            @pl.when(i < n)
            def _():
                gather_desc(i, b).wait()

                @pl.when(i + 1 < n)
                def _():
                    bn = (b + 1) % NBUF
                    wait_loads(i + 1, bn)
                    gather_desc(i + 1, bn).start(add=True)

                def rrow(r, _):
                    ebuf[b, r, pl.ds(0, 16)] = jnp.maximum(
                        ebuf[b, r, pl.ds(0, 16)], 0.0)
                    ebuf[b, r, pl.ds(16, 16)] = jnp.maximum(
                        ebuf[b, r, pl.ds(16, 16)], 0.0)
                    return _
                lax.fori_loop(0, K, rrow, None, unroll=8)

                @pl.when(i > 0)
                def _():
                    scatter_desc((b - 1) % NBUF).wait()

                @pl.when(i + 2 < n)
                def _():
                    start_loads(i + 2, (b + 2) % NBUF)

                scatter_desc(b).start(add=True)
        return _
    lax.fori_loop(0, (CH_A + NBUF - 1) // NBUF, group, None)
    scatter_desc(0).wait()
    plsc.subcore_barrier()
    for j in range(ROWS_PT // ZROWS):
        sl = pl.ds(base + j * ZROWS, ZROWS)
        pltpu.sync_copy(spacc.at[sl], out_ref.at[sl])


def _edge_body(hlo, hhi, elo, ehi, ei, agg_lo, agg_hi,
               sib, ebuf, zbuf, spacc, sem_l, sem_g, sem_s):
    c = lax.axis_index("c")
    s = lax.axis_index("s")

    @pl.when(c == 0)
    def _():
        _edge_half(s, hlo, elo, ei, agg_lo,
                   sib, ebuf, zbuf, spacc, sem_l, sem_g, sem_s)

    @pl.when(c == 1)
    def _():
        _edge_half(s, hhi, ehi, ei, agg_hi,
                   sib, ebuf, zbuf, spacc, sem_l, sem_g, sem_s)


def _edge_call(hlo, hhi, elo, ehi, edge_index):
    mesh = plsc.VectorSubcoreMesh(core_axis_name="c", subcore_axis_name="s")
    fn = pl.kernel(
        _edge_body,
        out_type=(
            jax.ShapeDtypeStruct((N_PAD, HH), f32),
            jax.ShapeDtypeStruct((N_PAD, HH), f32),
        ),
        mesh=mesh,
        scratch_types=[
            pltpu.VMEM((NBUF, 2, K), jnp.int32),
            pltpu.VMEM((NBUF, K, HH), f32),
            pltpu.VMEM((ZROWS, HH), f32),
            pltpu.VMEM_SHARED((N_PAD, HH), f32),
            pltpu.SemaphoreType.DMA,
            pltpu.SemaphoreType.DMA,
            pltpu.SemaphoreType.DMA,
        ],
        compiler_params=pltpu.CompilerParams(use_tc_tiling_on_sc=False),
    )
    return fn(hlo, hhi, elo, ehi, edge_index)


def kernel(feat, eweight, edge_index, Wn, We, params, Wo, bo):
    feat_p = jnp.pad(feat, ((0, 0), (0, H - feat.shape[1])))
    Wn_p = jnp.pad(Wn, ((0, H - Wn.shape[0]), (0, 0)))
    ew_p = jnp.pad(eweight, ((0, 0), (0, 8 - eweight.shape[1])))
    We_p = jnp.pad(We, ((0, 8 - We.shape[0]), (0, 0)))

    h_lo, h_hi = _embed2(feat_p, Wn_p, BN)
    e_lo, e_hi = _embed2(ew_p, We_p, BE)

    wo_r = Wo.reshape(1, H)
    bo_r = bo.reshape(1, 1)

    out = None
    for li, (W1, b1, W2, b2) in enumerate(params):
        agg_lo, agg_hi = _edge_call(h_lo, h_hi, e_lo, e_hi, edge_index)
        b1r = b1.reshape(1, H)
        b2r = b2.reshape(1, H)
        if li < 3:
            h_lo, h_hi = _mlp2(h_lo, h_hi, agg_lo, agg_hi, W1, b1r, W2, b2r)
        else:
            out = _mlp_readout(h_lo, h_hi, agg_lo, agg_hi,
                               W1, b1r, W2, b2r, wo_r, bo_r)
    return out


# speedup vs baseline: 6.0353x; 1.3732x over previous
import functools

import jax
import jax.numpy as jnp
from jax import lax
from jax.experimental import pallas as pl
from jax.experimental.pallas import tpu as pltpu
from jax.experimental.pallas import tpu_sc as plsc

f32 = jnp.float32

N = 50000
E = 800000
H = 64
HH = H // 2
PK = 16

N_PAD = 50176
NP = N_PAD // PK
EP = E // PK
NV = N // PK

BPN = 392
BPE = 2000

NSUB = 16
K = 128
EPT = 50048
CH_A = 391
CH_B = 385
NBUF = 3
ROWS_PT = N_PAD // NSUB
ZROWS = 392


def _kron16(w):
    return jnp.kron(jnp.eye(PK, dtype=f32), w)


def _embed_kernel(x_ref, wlo_ref, whi_ref, olo_ref, ohi_ref):
    x = x_ref[...]
    zlo = jnp.dot(x, wlo_ref[...], preferred_element_type=f32)
    zhi = jnp.dot(x, whi_ref[...], preferred_element_type=f32)
    olo_ref[...] = zlo.reshape(zlo.shape[0] * 512)
    ohi_ref[...] = zhi.reshape(zhi.shape[0] * 512)


def _embed(x, wlo, whi, blk):
    rows = x.shape[0]
    grid = rows // blk
    return pl.pallas_call(
        _embed_kernel,
        grid=(grid,),
        in_specs=[
            pl.BlockSpec((blk, x.shape[1]), lambda i: (i, 0)),
            pl.BlockSpec(wlo.shape, lambda i: (0, 0)),
            pl.BlockSpec(whi.shape, lambda i: (0, 0)),
        ],
        out_specs=[
            pl.BlockSpec((blk * 512,), lambda i: (i,)),
            pl.BlockSpec((blk * 512,), lambda i: (i,)),
        ],
        out_shape=[
            jax.ShapeDtypeStruct((rows * 512,), f32),
            jax.ShapeDtypeStruct((rows * 512,), f32),
        ],
    )(x, wlo, whi)


def _mlp_core(hl, hh, al, ah, kw, bias):
    (k11, k12, k21, k22, l11, l12, l21, l22) = kw
    (b1l, b1h, b2l, b2h) = bias
    zl = hl + al
    zh = hh + ah
    y1l = jnp.maximum(jnp.dot(zl, k11, preferred_element_type=f32)
                      + jnp.dot(zh, k21, preferred_element_type=f32)
                      + b1l, 0.0)
    y1h = jnp.maximum(jnp.dot(zl, k12, preferred_element_type=f32)
                      + jnp.dot(zh, k22, preferred_element_type=f32)
                      + b1h, 0.0)
    z2l = (jnp.dot(y1l, l11, preferred_element_type=f32)
           + jnp.dot(y1h, l21, preferred_element_type=f32) + b2l)
    z2h = (jnp.dot(y1l, l12, preferred_element_type=f32)
           + jnp.dot(y1h, l22, preferred_element_type=f32) + b2h)
    return jnp.maximum(z2l, 0.0) + hl, jnp.maximum(z2h, 0.0) + hh


def _mlp_kernel(hl_ref, hh_ref, al_ref, ah_ref,
                k11_ref, k12_ref, k21_ref, k22_ref,
                l11_ref, l12_ref, l21_ref, l22_ref,
                bb_ref, ol_ref, oh_ref):
    hl = hl_ref[...].reshape(BPN, 512)
    hh = hh_ref[...].reshape(BPN, 512)
    al = al_ref[...].reshape(BPN, 512)
    ah = ah_ref[...].reshape(BPN, 512)
    kw = (k11_ref[...], k12_ref[...], k21_ref[...], k22_ref[...],
          l11_ref[...], l12_ref[...], l21_ref[...], l22_ref[...])
    bias = (bb_ref[0:1, :], bb_ref[1:2, :], bb_ref[2:3, :], bb_ref[3:4, :])
    hnl, hnh = _mlp_core(hl, hh, al, ah, kw, bias)
    ol_ref[...] = hnl.reshape(BPN * 512)
    oh_ref[...] = hnh.reshape(BPN * 512)


def _mlp_specs():
    flat = pl.BlockSpec((BPN * 512,), lambda i: (i,))
    w = pl.BlockSpec((512, 512), lambda i: (0, 0))
    return ([flat, flat, flat, flat, w, w, w, w, w, w, w, w,
             pl.BlockSpec((4, 512), lambda i: (0, 0))], flat)


def _mlp(hl, hh, al, ah, kws, bb):
    in_specs, flat = _mlp_specs()
    return pl.pallas_call(
        _mlp_kernel,
        grid=(NP // BPN,),
        in_specs=in_specs,
        out_specs=[flat, flat],
        out_shape=[
            jax.ShapeDtypeStruct((N_PAD * HH,), f32),
            jax.ShapeDtypeStruct((N_PAD * HH,), f32),
        ],
    )(hl, hh, al, ah, *kws, bb)


def _mlp_readout_kernel(hl_ref, hh_ref, al_ref, ah_ref,
                        k11_ref, k12_ref, k21_ref, k22_ref,
                        l11_ref, l12_ref, l21_ref, l22_ref,
                        bb_ref, fold_ref, wo_ref, bo_ref,
                        out_ref, accl_ref, acch_ref):
    i = pl.program_id(0)
    hl = hl_ref[...].reshape(BPN, 512)
    hh = hh_ref[...].reshape(BPN, 512)
    al = al_ref[...].reshape(BPN, 512)
    ah = ah_ref[...].reshape(BPN, 512)
    kw = (k11_ref[...], k12_ref[...], k21_ref[...], k22_ref[...],
          l11_ref[...], l12_ref[...], l21_ref[...], l22_ref[...])
    bias = (bb_ref[0:1, :], bb_ref[1:2, :], bb_ref[2:3, :], bb_ref[3:4, :])
    hnl, hnh = _mlp_core(hl, hh, al, ah, kw, bias)
    row = lax.broadcasted_iota(jnp.int32, (BPN, 1), 0) + i * BPN
    valid = row < NV
    pl_ = jnp.sum(jnp.where(valid, hnl, 0.0), axis=0, keepdims=True)
    ph_ = jnp.sum(jnp.where(valid, hnh, 0.0), axis=0, keepdims=True)

    @pl.when(i == 0)
    def _():
        accl_ref[...] = pl_
        acch_ref[...] = ph_

    @pl.when(i > 0)
    def _():
        accl_ref[...] = accl_ref[...] + pl_
        acch_ref[...] = acch_ref[...] + ph_

    @pl.when(i == pl.num_programs(0) - 1)
    def _():
        tl = jnp.dot(accl_ref[...], fold_ref[...],
                     preferred_element_type=f32)
        th = jnp.dot(acch_ref[...], fold_ref[...],
                     preferred_element_type=f32)
        out_ref[...] = (jnp.sum(tl * wo_ref[0:1, :], axis=1, keepdims=True)
                        + jnp.sum(th * wo_ref[1:2, :], axis=1, keepdims=True)
                        + bo_ref[...])


def _mlp_readout(hl, hh, al, ah, kws, bb, fold, wo2, bo_r):
    in_specs, _ = _mlp_specs()
    in_specs = in_specs + [
        pl.BlockSpec((512, HH), lambda i: (0, 0)),
        pl.BlockSpec((2, HH), lambda i: (0, 0)),
        pl.BlockSpec((1, 1), lambda i: (0, 0)),
    ]
    return pl.pallas_call(
        _mlp_readout_kernel,
        grid=(NP // BPN,),
        in_specs=in_specs,
        out_specs=pl.BlockSpec((1, 1), lambda i: (0, 0)),
        out_shape=jax.ShapeDtypeStruct((1, 1), f32),
        scratch_shapes=[pltpu.VMEM((1, 512), f32), pltpu.VMEM((1, 512), f32)],
    )(hl, hh, al, ah, *kws, bb, fold, wo2, bo_r)


def _edge_half(s, h_ref, e_ref, ei_ref, out_ref,
               sib, ebuf, zbuf, spacc, sem_l, sem_g, sem_s):
    def zrow(i, _):
        zbuf[i, pl.ds(0, 16)] = jnp.zeros((16,), f32)
        zbuf[i, pl.ds(16, 16)] = jnp.zeros((16,), f32)
        return _
    lax.fori_loop(0, ZROWS, zrow, None, unroll=4)
    base = s * ROWS_PT
    for j in range(ROWS_PT // ZROWS):
        pltpu.sync_copy(zbuf, spacc.at[pl.ds(base + j * ZROWS, ZROWS)])
    plsc.subcore_barrier()

    n = jnp.where(s < NSUB - 1, CH_A, CH_B)
    ebase = s * EPT

    def load_copies(i, b):
        lo = ebase + i * K
        return (pltpu.make_async_copy(e_ref.at[pl.ds(lo, K)], ebuf.at[b],
                                      sem_l),
                pltpu.make_async_copy(ei_ref.at[pl.ds(lo, K)], sib.at[b, 0],
                                      sem_l),
                pltpu.make_async_copy(ei_ref.at[pl.ds(E + lo, K)],
                                      sib.at[b, 1], sem_l))

    def start_loads(i, b):
        for d in load_copies(i, b):
            d.start()

    def wait_loads(i, b):
        for d in load_copies(i, b):
            d.wait()

    def gather_desc(i, b):
        return pltpu.make_async_copy(h_ref.at[sib.at[b, 0]], ebuf.at[b],
                                     sem_g)

    def scatter_desc(b):
        return pltpu.make_async_copy(ebuf.at[b], spacc.at[sib.at[b, 1]],
                                     sem_s)

    start_loads(0, 0)
    wait_loads(0, 0)
    gather_desc(0, 0).start(add=True)
    start_loads(1, 1)

    def group(g, carry):
        for b in range(NBUF):
            i = g * NBUF + b

            @pl.when(i < n)
            def _():
                gather_desc(i, b).wait()

                @pl.when(i + 1 < n)
                def _():
                    bn = (b + 1) % NBUF
                    wait_loads(i + 1, bn)
                    gather_desc(i + 1, bn).start(add=True)

                def rrow(r, _):
                    ebuf[b, r, pl.ds(0, 16)] = jnp.maximum(
                        ebuf[b, r, pl.ds(0, 16)], 0.0)
                    ebuf[b, r, pl.ds(16, 16)] = jnp.maximum(
                        ebuf[b, r, pl.ds(16, 16)], 0.0)
                    return _
                lax.fori_loop(0, K, rrow, None, unroll=8)

                @pl.when(i > 0)
                def _():
                    scatter_desc((b - 1) % NBUF).wait()

                @pl.when(i + 2 < n)
                def _():
                    start_loads(i + 2, (b + 2) % NBUF)

                scatter_desc(b).start(add=True)
        return carry
    lax.fori_loop(0, (CH_A + NBUF - 1) // NBUF, group, None)
    scatter_desc(0).wait()
    plsc.subcore_barrier()
    for j in range(ROWS_PT // ZROWS):
        sl = pl.ds(base + j * ZROWS, ZROWS)
        pltpu.sync_copy(spacc.at[sl], out_ref.at[sl])


def _edge_body(hlo, hhi, elo, ehi, ei, agg_lo, agg_hi,
               sib, ebuf, zbuf, spacc, sem_l, sem_g, sem_s):
    c = lax.axis_index("c")
    s = lax.axis_index("s")

    @pl.when(c == 0)
    def _():
        _edge_half(s, hlo, elo, ei, agg_lo,
                   sib, ebuf, zbuf, spacc, sem_l, sem_g, sem_s)

    @pl.when(c == 1)
    def _():
        _edge_half(s, hhi, ehi, ei, agg_hi,
                   sib, ebuf, zbuf, spacc, sem_l, sem_g, sem_s)


def _edge_call(hlo, hhi, elo, ehi, ei_flat):
    mesh = plsc.VectorSubcoreMesh(core_axis_name="c", subcore_axis_name="s")
    fn = pl.kernel(
        _edge_body,
        out_type=(
            jax.ShapeDtypeStruct((N_PAD, HH), f32),
            jax.ShapeDtypeStruct((N_PAD, HH), f32),
        ),
        mesh=mesh,
        scratch_types=[
            pltpu.VMEM((NBUF, 2, K), jnp.int32),
            pltpu.VMEM((NBUF, K, HH), f32),
            pltpu.VMEM((ZROWS, HH), f32),
            pltpu.VMEM_SHARED((N_PAD, HH), f32),
            pltpu.SemaphoreType.DMA,
            pltpu.SemaphoreType.DMA,
            pltpu.SemaphoreType.DMA,
        ],
        compiler_params=pltpu.CompilerParams(use_tc_tiling_on_sc=False),
    )
    return fn(hlo, hhi, elo, ehi, ei_flat)


def kernel(feat, eweight, edge_index, Wn, We, params, Wo, bo):
    Wn_p = jnp.pad(Wn, ((0, H - Wn.shape[0]), (0, 0)))
    We_p = jnp.pad(We, ((0, 8 - We.shape[0]), (0, 0)))
    wn_lo = _kron16(Wn_p[:, :HH])
    wn_hi = _kron16(Wn_p[:, HH:])
    we_lo = _kron16(We_p[:, :HH])
    we_hi = _kron16(We_p[:, HH:])
    fold = jnp.kron(jnp.ones((PK, 1), f32), jnp.eye(HH, dtype=f32))
    wo2 = Wo.reshape(2, HH)
    bo_r = bo.reshape(1, 1)

    featp = jnp.pad(feat, ((0, N_PAD - N), (0, H - feat.shape[1])))
    featp = featp.reshape(NP, PK * H)
    ewp = jnp.pad(eweight, ((0, 0), (0, 8 - eweight.shape[1])))
    ewp = ewp.reshape(EP, PK * 8)
    ei_flat = edge_index.reshape(2 * E)

    h_lo, h_hi = _embed(featp, wn_lo, wn_hi, BPN)
    e_lo, e_hi = _embed(ewp, we_lo, we_hi, BPE)
    e_lo2 = e_lo.reshape(E, HH)
    e_hi2 = e_hi.reshape(E, HH)

    out = None
    for li, (W1, b1, W2, b2) in enumerate(params):
        kws = (_kron16(W1[:HH, :HH]), _kron16(W1[:HH, HH:]),
               _kron16(W1[HH:, :HH]), _kron16(W1[HH:, HH:]),
               _kron16(W2[:HH, :HH]), _kron16(W2[:HH, HH:]),
               _kron16(W2[HH:, :HH]), _kron16(W2[HH:, HH:]))
        bb = jnp.stack([jnp.tile(b1[:HH], PK), jnp.tile(b1[HH:], PK),
                        jnp.tile(b2[:HH], PK), jnp.tile(b2[HH:], PK)])
        agg_lo, agg_hi = _edge_call(h_lo.reshape(N_PAD, HH),
                                    h_hi.reshape(N_PAD, HH),
                                    e_lo2, e_hi2, ei_flat)
        al = agg_lo.reshape(N_PAD * HH)
        ah = agg_hi.reshape(N_PAD * HH)
        if li < 3:
            h_lo, h_hi = _mlp(h_lo, h_hi, al, ah, kws, bb)
        else:
            out = _mlp_readout(h_lo, h_hi, al, ah, kws, bb, fold, wo2, bo_r)
    return out


# speedup vs baseline: 6.4815x; 1.0739x over previous
import functools

import jax
import jax.numpy as jnp
from jax import lax
from jax.experimental import pallas as pl
from jax.experimental.pallas import tpu as pltpu
from jax.experimental.pallas import tpu_sc as plsc

f32 = jnp.float32

N = 50000
E = 800000
H = 64
HH = H // 2
PK = 16

N_PAD = 50176
NP = N_PAD // PK
EP = E // PK
NV = N // PK

BPN = 392
BPE = 2000

NSUB = 16
K = 128
EPT = 50048
CH_A = 391
CH_B = 385
NBUF = 4
ROWS_PT = N_PAD // NSUB
ZROWS = 392


def _kron16(w):
    return jnp.kron(jnp.eye(PK, dtype=f32), w)


def _embed_kernel(x_ref, wlo_ref, whi_ref, olo_ref, ohi_ref):
    x = x_ref[...]
    zlo = jnp.dot(x, wlo_ref[...], preferred_element_type=f32)
    zhi = jnp.dot(x, whi_ref[...], preferred_element_type=f32)
    olo_ref[...] = zlo.reshape(zlo.shape[0] * 512)
    ohi_ref[...] = zhi.reshape(zhi.shape[0] * 512)


def _embed(x, wlo, whi, blk):
    rows = x.shape[0]
    grid = rows // blk
    return pl.pallas_call(
        _embed_kernel,
        grid=(grid,),
        in_specs=[
            pl.BlockSpec((blk, x.shape[1]), lambda i: (i, 0)),
            pl.BlockSpec(wlo.shape, lambda i: (0, 0)),
            pl.BlockSpec(whi.shape, lambda i: (0, 0)),
        ],
        out_specs=[
            pl.BlockSpec((blk * 512,), lambda i: (i,)),
            pl.BlockSpec((blk * 512,), lambda i: (i,)),
        ],
        out_shape=[
            jax.ShapeDtypeStruct((rows * 512,), f32),
            jax.ShapeDtypeStruct((rows * 512,), f32),
        ],
    )(x, wlo, whi)


def _mlp_core(hl, hh, al, ah, kw, bias):
    (k11, k12, k21, k22, l11, l12, l21, l22) = kw
    (b1l, b1h, b2l, b2h) = bias
    zl = hl + al
    zh = hh + ah
    y1l = jnp.maximum(jnp.dot(zl, k11, preferred_element_type=f32)
                      + jnp.dot(zh, k21, preferred_element_type=f32)
                      + b1l, 0.0)
    y1h = jnp.maximum(jnp.dot(zl, k12, preferred_element_type=f32)
                      + jnp.dot(zh, k22, preferred_element_type=f32)
                      + b1h, 0.0)
    z2l = (jnp.dot(y1l, l11, preferred_element_type=f32)
           + jnp.dot(y1h, l21, preferred_element_type=f32) + b2l)
    z2h = (jnp.dot(y1l, l12, preferred_element_type=f32)
           + jnp.dot(y1h, l22, preferred_element_type=f32) + b2h)
    return jnp.maximum(z2l, 0.0) + hl, jnp.maximum(z2h, 0.0) + hh


def _mlp_kernel(hl_ref, hh_ref, al_ref, ah_ref,
                k11_ref, k12_ref, k21_ref, k22_ref,
                l11_ref, l12_ref, l21_ref, l22_ref,
                bb_ref, ol_ref, oh_ref):
    hl = hl_ref[...].reshape(BPN, 512)
    hh = hh_ref[...].reshape(BPN, 512)
    al = al_ref[...].reshape(BPN, 512)
    ah = ah_ref[...].reshape(BPN, 512)
    kw = (k11_ref[...], k12_ref[...], k21_ref[...], k22_ref[...],
          l11_ref[...], l12_ref[...], l21_ref[...], l22_ref[...])
    bias = (bb_ref[0:1, :], bb_ref[1:2, :], bb_ref[2:3, :], bb_ref[3:4, :])
    hnl, hnh = _mlp_core(hl, hh, al, ah, kw, bias)
    ol_ref[...] = hnl.reshape(BPN * 512)
    oh_ref[...] = hnh.reshape(BPN * 512)


def _mlp_specs():
    flat = pl.BlockSpec((BPN * 512,), lambda i: (i,))
    w = pl.BlockSpec((512, 512), lambda i: (0, 0))
    return ([flat, flat, flat, flat, w, w, w, w, w, w, w, w,
             pl.BlockSpec((4, 512), lambda i: (0, 0))], flat)


def _mlp(hl, hh, al, ah, kws, bb):
    in_specs, flat = _mlp_specs()
    return pl.pallas_call(
        _mlp_kernel,
        grid=(NP // BPN,),
        in_specs=in_specs,
        out_specs=[flat, flat],
        out_shape=[
            jax.ShapeDtypeStruct((N_PAD * HH,), f32),
            jax.ShapeDtypeStruct((N_PAD * HH,), f32),
        ],
    )(hl, hh, al, ah, *kws, bb)


def _mlp_readout_kernel(hl_ref, hh_ref, al_ref, ah_ref,
                        k11_ref, k12_ref, k21_ref, k22_ref,
                        l11_ref, l12_ref, l21_ref, l22_ref,
                        bb_ref, fold_ref, wo_ref, bo_ref,
                        out_ref, accl_ref, acch_ref):
    i = pl.program_id(0)
    hl = hl_ref[...].reshape(BPN, 512)
    hh = hh_ref[...].reshape(BPN, 512)
    al = al_ref[...].reshape(BPN, 512)
    ah = ah_ref[...].reshape(BPN, 512)
    kw = (k11_ref[...], k12_ref[...], k21_ref[...], k22_ref[...],
          l11_ref[...], l12_ref[...], l21_ref[...], l22_ref[...])
    bias = (bb_ref[0:1, :], bb_ref[1:2, :], bb_ref[2:3, :], bb_ref[3:4, :])
    hnl, hnh = _mlp_core(hl, hh, al, ah, kw, bias)
    row = lax.broadcasted_iota(jnp.int32, (BPN, 1), 0) + i * BPN
    valid = row < NV
    pl_ = jnp.sum(jnp.where(valid, hnl, 0.0), axis=0, keepdims=True)
    ph_ = jnp.sum(jnp.where(valid, hnh, 0.0), axis=0, keepdims=True)

    @pl.when(i == 0)
    def _():
        accl_ref[...] = pl_
        acch_ref[...] = ph_

    @pl.when(i > 0)
    def _():
        accl_ref[...] = accl_ref[...] + pl_
        acch_ref[...] = acch_ref[...] + ph_

    @pl.when(i == pl.num_programs(0) - 1)
    def _():
        tl = jnp.dot(accl_ref[...], fold_ref[...],
                     preferred_element_type=f32)
        th = jnp.dot(acch_ref[...], fold_ref[...],
                     preferred_element_type=f32)
        out_ref[...] = (jnp.sum(tl * wo_ref[0:1, :], axis=1, keepdims=True)
                        + jnp.sum(th * wo_ref[1:2, :], axis=1, keepdims=True)
                        + bo_ref[...])


def _mlp_readout(hl, hh, al, ah, kws, bb, fold, wo2, bo_r):
    in_specs, _ = _mlp_specs()
    in_specs = in_specs + [
        pl.BlockSpec((512, HH), lambda i: (0, 0)),
        pl.BlockSpec((2, HH), lambda i: (0, 0)),
        pl.BlockSpec((1, 1), lambda i: (0, 0)),
    ]
    return pl.pallas_call(
        _mlp_readout_kernel,
        grid=(NP // BPN,),
        in_specs=in_specs,
        out_specs=pl.BlockSpec((1, 1), lambda i: (0, 0)),
        out_shape=jax.ShapeDtypeStruct((1, 1), f32),
        scratch_shapes=[pltpu.VMEM((1, 512), f32), pltpu.VMEM((1, 512), f32)],
    )(hl, hh, al, ah, *kws, bb, fold, wo2, bo_r)


def _edge_half(s, h_ref, e_ref, ei_ref, out_ref,
               sib, ebuf, zbuf, spacc, sem_l, sem_g, sem_s):
    def zrow(i, _):
        zbuf[i, pl.ds(0, 16)] = jnp.zeros((16,), f32)
        zbuf[i, pl.ds(16, 16)] = jnp.zeros((16,), f32)
        return _
    lax.fori_loop(0, ZROWS, zrow, None, unroll=4)
    base = s * ROWS_PT
    for j in range(ROWS_PT // ZROWS):
        pltpu.sync_copy(zbuf, spacc.at[pl.ds(base + j * ZROWS, ZROWS)])
    plsc.subcore_barrier()

    n = jnp.where(s < NSUB - 1, CH_A, CH_B)
    ebase = s * EPT

    def load_copies(i, b):
        lo = ebase + i * K
        return (pltpu.make_async_copy(e_ref.at[pl.ds(lo, K)], ebuf.at[b],
                                      sem_l),
                pltpu.make_async_copy(ei_ref.at[pl.ds(lo, K)], sib.at[b, 0],
                                      sem_l),
                pltpu.make_async_copy(ei_ref.at[pl.ds(E + lo, K)],
                                      sib.at[b, 1], sem_l))

    def start_loads(i, b):
        for d in load_copies(i, b):
            d.start()

    def wait_loads(i, b):
        for d in load_copies(i, b):
            d.wait()

    def gather_desc(i, b):
        return pltpu.make_async_copy(h_ref.at[sib.at[b, 0]], ebuf.at[b],
                                     sem_g)

    def scatter_desc(b):
        return pltpu.make_async_copy(ebuf.at[b], spacc.at[sib.at[b, 1]],
                                     sem_s)

    start_loads(0, 0)
    wait_loads(0, 0)
    gather_desc(0, 0).start(add=True)
    start_loads(1, 1)
    wait_loads(1, 1)
    gather_desc(1, 1).start(add=True)
    start_loads(2, 2)

    def group(g, carry):
        for b in range(NBUF):
            i = g * NBUF + b

            @pl.when(i < n)
            def _():
                gather_desc(i, b).wait()

                def rrow(r, _):
                    ebuf[b, r, pl.ds(0, 16)] = jnp.maximum(
                        ebuf[b, r, pl.ds(0, 16)], 0.0)
                    ebuf[b, r, pl.ds(16, 16)] = jnp.maximum(
                        ebuf[b, r, pl.ds(16, 16)], 0.0)
                    return _
                lax.fori_loop(0, K, rrow, None, unroll=8)

                @pl.when(i > 0)
                def _():
                    scatter_desc((b - 1) % NBUF).wait()

                @pl.when(i + 2 < n)
                def _():
                    bn = (b + 2) % NBUF
                    wait_loads(i + 2, bn)
                    gather_desc(i + 2, bn).start(add=True)

                @pl.when(i + 3 < n)
                def _():
                    start_loads(i + 3, (b + 3) % NBUF)

                scatter_desc(b).start(add=True)
        return carry
    lax.fori_loop(0, (CH_A + NBUF - 1) // NBUF, group, None)
    @pl.when(s < NSUB - 1)
    def _():
        scatter_desc(2).wait()

    @pl.when(s == NSUB - 1)
    def _():
        scatter_desc(0).wait()
    plsc.subcore_barrier()
    for j in range(ROWS_PT // ZROWS):
        sl = pl.ds(base + j * ZROWS, ZROWS)
        pltpu.sync_copy(spacc.at[sl], out_ref.at[sl])


def _edge_body(hlo, hhi, elo, ehi, ei, agg_lo, agg_hi,
               sib, ebuf, zbuf, spacc, sem_l, sem_g, sem_s):
    c = lax.axis_index("c")
    s = lax.axis_index("s")

    @pl.when(c == 0)
    def _():
        _edge_half(s, hlo, elo, ei, agg_lo,
                   sib, ebuf, zbuf, spacc, sem_l, sem_g, sem_s)

    @pl.when(c == 1)
    def _():
        _edge_half(s, hhi, ehi, ei, agg_hi,
                   sib, ebuf, zbuf, spacc, sem_l, sem_g, sem_s)


def _edge_call(hlo, hhi, elo, ehi, ei_flat):
    mesh = plsc.VectorSubcoreMesh(core_axis_name="c", subcore_axis_name="s")
    fn = pl.kernel(
        _edge_body,
        out_type=(
            jax.ShapeDtypeStruct((N_PAD, HH), f32),
            jax.ShapeDtypeStruct((N_PAD, HH), f32),
        ),
        mesh=mesh,
        scratch_types=[
            pltpu.VMEM((NBUF, 2, K), jnp.int32),
            pltpu.VMEM((NBUF, K, HH), f32),
            pltpu.VMEM((ZROWS, HH), f32),
            pltpu.VMEM_SHARED((N_PAD, HH), f32),
            pltpu.SemaphoreType.DMA,
            pltpu.SemaphoreType.DMA,
            pltpu.SemaphoreType.DMA,
        ],
        compiler_params=pltpu.CompilerParams(use_tc_tiling_on_sc=False),
    )
    return fn(hlo, hhi, elo, ehi, ei_flat)


def kernel(feat, eweight, edge_index, Wn, We, params, Wo, bo):
    Wn_p = jnp.pad(Wn, ((0, H - Wn.shape[0]), (0, 0)))
    We_p = jnp.pad(We, ((0, 8 - We.shape[0]), (0, 0)))
    wn_lo = _kron16(Wn_p[:, :HH])
    wn_hi = _kron16(Wn_p[:, HH:])
    we_lo = _kron16(We_p[:, :HH])
    we_hi = _kron16(We_p[:, HH:])
    fold = jnp.kron(jnp.ones((PK, 1), f32), jnp.eye(HH, dtype=f32))
    wo2 = Wo.reshape(2, HH)
    bo_r = bo.reshape(1, 1)

    featp = jnp.pad(feat, ((0, N_PAD - N), (0, H - feat.shape[1])))
    featp = featp.reshape(NP, PK * H)
    ewp = jnp.pad(eweight, ((0, 0), (0, 8 - eweight.shape[1])))
    ewp = ewp.reshape(E * 8).reshape(EP, PK * 8)
    ei_flat = edge_index.reshape(2 * E)

    h_lo, h_hi = _embed(featp, wn_lo, wn_hi, BPN)
    e_lo, e_hi = _embed(ewp, we_lo, we_hi, BPE)
    e_lo2 = e_lo.reshape(E, HH)
    e_hi2 = e_hi.reshape(E, HH)

    out = None
    for li, (W1, b1, W2, b2) in enumerate(params):
        kws = (_kron16(W1[:HH, :HH]), _kron16(W1[:HH, HH:]),
               _kron16(W1[HH:, :HH]), _kron16(W1[HH:, HH:]),
               _kron16(W2[:HH, :HH]), _kron16(W2[:HH, HH:]),
               _kron16(W2[HH:, :HH]), _kron16(W2[HH:, HH:]))
        bb = jnp.stack([jnp.tile(b1[:HH], PK), jnp.tile(b1[HH:], PK),
                        jnp.tile(b2[:HH], PK), jnp.tile(b2[HH:], PK)])
        agg_lo, agg_hi = _edge_call(h_lo.reshape(N_PAD, HH),
                                    h_hi.reshape(N_PAD, HH),
                                    e_lo2, e_hi2, ei_flat)
        al = agg_lo.reshape(N_PAD * HH)
        ah = agg_hi.reshape(N_PAD * HH)
        if li < 3:
            h_lo, h_hi = _mlp(h_lo, h_hi, al, ah, kws, bb)
        else:
            out = _mlp_readout(h_lo, h_hi, al, ah, kws, bb, fold, wo2, bo_r)
    return out


# speedup vs baseline: 7.0802x; 1.0924x over previous
import functools

import jax
import jax.numpy as jnp
from jax import lax
from jax.experimental import pallas as pl
from jax.experimental.pallas import tpu as pltpu
from jax.experimental.pallas import tpu_sc as plsc

f32 = jnp.float32

N = 50000
E = 800000
H = 64
HH = H // 2
PK = 16

N_PAD = 50176
NP = N_PAD // PK
EP = E // PK
NV = N // PK

BPN = 392
BPE = 1000

NSUB = 16
K = 128
EPT = 50048
CH_A = 391
CH_B = 385
NBUF = 4
ROWS_PT = N_PAD // NSUB
ZROWS = 392


def _kron16(w):
    return jnp.kron(jnp.eye(PK, dtype=f32), w)


def _embed_kernel(x_ref, wlo_ref, whi_ref, olo_ref, ohi_ref):
    x = x_ref[...]
    zlo = jnp.dot(x, wlo_ref[...], preferred_element_type=f32)
    zhi = jnp.dot(x, whi_ref[...], preferred_element_type=f32)
    olo_ref[...] = zlo.reshape(zlo.shape[0] * zlo.shape[1])
    ohi_ref[...] = zhi.reshape(zhi.shape[0] * zhi.shape[1])


def _embed(x, wlo, whi, blk):
    rows = x.shape[0]
    outw = wlo.shape[1]
    grid = rows // blk
    return pl.pallas_call(
        _embed_kernel,
        grid=(grid,),
        in_specs=[
            pl.BlockSpec((blk, x.shape[1]), lambda i: (i, 0)),
            pl.BlockSpec(wlo.shape, lambda i: (0, 0)),
            pl.BlockSpec(whi.shape, lambda i: (0, 0)),
        ],
        out_specs=[
            pl.BlockSpec((blk * outw,), lambda i: (i,)),
            pl.BlockSpec((blk * outw,), lambda i: (i,)),
        ],
        out_shape=[
            jax.ShapeDtypeStruct((rows * outw,), f32),
            jax.ShapeDtypeStruct((rows * outw,), f32),
        ],
    )(x, wlo, whi)


def _mlp_core(hl, hh, al, ah, kw, bias):
    (k11, k12, k21, k22, l11, l12, l21, l22) = kw
    (b1l, b1h, b2l, b2h) = bias
    zl = hl + al
    zh = hh + ah
    y1l = jnp.maximum(jnp.dot(zl, k11, preferred_element_type=f32)
                      + jnp.dot(zh, k21, preferred_element_type=f32)
                      + b1l, 0.0)
    y1h = jnp.maximum(jnp.dot(zl, k12, preferred_element_type=f32)
                      + jnp.dot(zh, k22, preferred_element_type=f32)
                      + b1h, 0.0)
    z2l = (jnp.dot(y1l, l11, preferred_element_type=f32)
           + jnp.dot(y1h, l21, preferred_element_type=f32) + b2l)
    z2h = (jnp.dot(y1l, l12, preferred_element_type=f32)
           + jnp.dot(y1h, l22, preferred_element_type=f32) + b2h)
    return jnp.maximum(z2l, 0.0) + hl, jnp.maximum(z2h, 0.0) + hh


def _mlp_kernel(hl_ref, hh_ref, al_ref, ah_ref,
                k11_ref, k12_ref, k21_ref, k22_ref,
                l11_ref, l12_ref, l21_ref, l22_ref,
                bb_ref, ol_ref, oh_ref):
    hl = hl_ref[...].reshape(BPN, 512)
    hh = hh_ref[...].reshape(BPN, 512)
    al = al_ref[...].reshape(BPN, 512)
    ah = ah_ref[...].reshape(BPN, 512)
    kw = (k11_ref[...], k12_ref[...], k21_ref[...], k22_ref[...],
          l11_ref[...], l12_ref[...], l21_ref[...], l22_ref[...])
    bias = (bb_ref[0:1, :], bb_ref[1:2, :], bb_ref[2:3, :], bb_ref[3:4, :])
    hnl, hnh = _mlp_core(hl, hh, al, ah, kw, bias)
    ol_ref[...] = hnl.reshape(BPN * 512)
    oh_ref[...] = hnh.reshape(BPN * 512)


def _mlp_specs():
    flat = pl.BlockSpec((BPN * 512,), lambda i: (i,))
    w = pl.BlockSpec((512, 512), lambda i: (0, 0))
    return ([flat, flat, flat, flat, w, w, w, w, w, w, w, w,
             pl.BlockSpec((4, 512), lambda i: (0, 0))], flat)


def _mlp(hl, hh, al, ah, kws, bb):
    in_specs, flat = _mlp_specs()
    return pl.pallas_call(
        _mlp_kernel,
        grid=(NP // BPN,),
        in_specs=in_specs,
        out_specs=[flat, flat],
        out_shape=[
            jax.ShapeDtypeStruct((N_PAD * HH,), f32),
            jax.ShapeDtypeStruct((N_PAD * HH,), f32),
        ],
    )(hl, hh, al, ah, *kws, bb)


def _mlp_readout_kernel(hl_ref, hh_ref, al_ref, ah_ref,
                        k11_ref, k12_ref, k21_ref, k22_ref,
                        l11_ref, l12_ref, l21_ref, l22_ref,
                        bb_ref, fold_ref, wo_ref, bo_ref,
                        out_ref, accl_ref, acch_ref):
    i = pl.program_id(0)
    hl = hl_ref[...].reshape(BPN, 512)
    hh = hh_ref[...].reshape(BPN, 512)
    al = al_ref[...].reshape(BPN, 512)
    ah = ah_ref[...].reshape(BPN, 512)
    kw = (k11_ref[...], k12_ref[...], k21_ref[...], k22_ref[...],
          l11_ref[...], l12_ref[...], l21_ref[...], l22_ref[...])
    bias = (bb_ref[0:1, :], bb_ref[1:2, :], bb_ref[2:3, :], bb_ref[3:4, :])
    hnl, hnh = _mlp_core(hl, hh, al, ah, kw, bias)
    row = lax.broadcasted_iota(jnp.int32, (BPN, 1), 0) + i * BPN
    valid = row < NV
    pl_ = jnp.sum(jnp.where(valid, hnl, 0.0), axis=0, keepdims=True)
    ph_ = jnp.sum(jnp.where(valid, hnh, 0.0), axis=0, keepdims=True)

    @pl.when(i == 0)
    def _():
        accl_ref[...] = pl_
        acch_ref[...] = ph_

    @pl.when(i > 0)
    def _():
        accl_ref[...] = accl_ref[...] + pl_
        acch_ref[...] = acch_ref[...] + ph_

    @pl.when(i == pl.num_programs(0) - 1)
    def _():
        tl = jnp.dot(accl_ref[...], fold_ref[...],
                     preferred_element_type=f32)
        th = jnp.dot(acch_ref[...], fold_ref[...],
                     preferred_element_type=f32)
        out_ref[...] = (jnp.sum(tl * wo_ref[0:1, :], axis=1, keepdims=True)
                        + jnp.sum(th * wo_ref[1:2, :], axis=1, keepdims=True)
                        + bo_ref[...])


def _mlp_readout(hl, hh, al, ah, kws, bb, fold, wo2, bo_r):
    in_specs, _ = _mlp_specs()
    in_specs = in_specs + [
        pl.BlockSpec((512, HH), lambda i: (0, 0)),
        pl.BlockSpec((2, HH), lambda i: (0, 0)),
        pl.BlockSpec((1, 1), lambda i: (0, 0)),
    ]
    return pl.pallas_call(
        _mlp_readout_kernel,
        grid=(NP // BPN,),
        in_specs=in_specs,
        out_specs=pl.BlockSpec((1, 1), lambda i: (0, 0)),
        out_shape=jax.ShapeDtypeStruct((1, 1), f32),
        scratch_shapes=[pltpu.VMEM((1, 512), f32), pltpu.VMEM((1, 512), f32)],
    )(hl, hh, al, ah, *kws, bb, fold, wo2, bo_r)


def _edge_half(s, h_ref, e_ref, ei_ref, out_ref,
               sib, ebuf, zbuf, spacc, sem_l, sem_g, sem_s):
    def zrow(i, _):
        zbuf[i, pl.ds(0, 16)] = jnp.zeros((16,), f32)
        zbuf[i, pl.ds(16, 16)] = jnp.zeros((16,), f32)
        return _
    lax.fori_loop(0, ZROWS, zrow, None, unroll=4)
    base = s * ROWS_PT
    for j in range(ROWS_PT // ZROWS):
        pltpu.sync_copy(zbuf, spacc.at[pl.ds(base + j * ZROWS, ZROWS)])
    plsc.subcore_barrier()

    n = jnp.where(s < NSUB - 1, CH_A, CH_B)
    ebase = s * EPT

    def load_copies(i, b):
        lo = ebase + i * K
        return (pltpu.make_async_copy(e_ref.at[pl.ds(lo, K)], ebuf.at[b],
                                      sem_l),
                pltpu.make_async_copy(ei_ref.at[pl.ds(lo, K)], sib.at[b, 0],
                                      sem_l),
                pltpu.make_async_copy(ei_ref.at[pl.ds(E + lo, K)],
                                      sib.at[b, 1], sem_l))

    def start_loads(i, b):
        for d in load_copies(i, b):
            d.start()

    def wait_loads(i, b):
        for d in load_copies(i, b):
            d.wait()

    def gather_desc(i, b):
        return pltpu.make_async_copy(h_ref.at[sib.at[b, 0]], ebuf.at[b],
                                     sem_g)

    def scatter_desc(b):
        return pltpu.make_async_copy(ebuf.at[b], spacc.at[sib.at[b, 1]],
                                     sem_s)

    start_loads(0, 0)
    wait_loads(0, 0)
    gather_desc(0, 0).start(add=True)
    start_loads(1, 1)
    wait_loads(1, 1)
    gather_desc(1, 1).start(add=True)
    start_loads(2, 2)

    def group(g, carry):
        for b in range(NBUF):
            i = g * NBUF + b

            @pl.when(i < n)
            def _():
                gather_desc(i, b).wait()

                def rrow(r, _):
                    ebuf[b, r, pl.ds(0, 16)] = jnp.maximum(
                        ebuf[b, r, pl.ds(0, 16)], 0.0)
                    ebuf[b, r, pl.ds(16, 16)] = jnp.maximum(
                        ebuf[b, r, pl.ds(16, 16)], 0.0)
                    return _
                lax.fori_loop(0, K, rrow, None, unroll=8)

                @pl.when(i > 0)
                def _():
                    scatter_desc((b - 1) % NBUF).wait()

                @pl.when(i + 2 < n)
                def _():
                    bn = (b + 2) % NBUF
                    wait_loads(i + 2, bn)
                    gather_desc(i + 2, bn).start(add=True)

                @pl.when(i + 3 < n)
                def _():
                    start_loads(i + 3, (b + 3) % NBUF)

                scatter_desc(b).start(add=True)
        return carry
    lax.fori_loop(0, (CH_A + NBUF - 1) // NBUF, group, None)
    @pl.when(s < NSUB - 1)
    def _():
        scatter_desc(2).wait()

    @pl.when(s == NSUB - 1)
    def _():
        scatter_desc(0).wait()
    plsc.subcore_barrier()
    for j in range(ROWS_PT // ZROWS):
        sl = pl.ds(base + j * ZROWS, ZROWS)
        pltpu.sync_copy(spacc.at[sl], out_ref.at[sl])


def _edge_body(hlo, hhi, elo, ehi, ei, agg_lo, agg_hi,
               sib, ebuf, zbuf, spacc, sem_l, sem_g, sem_s):
    c = lax.axis_index("c")
    s = lax.axis_index("s")

    @pl.when(c == 0)
    def _():
        _edge_half(s, hlo, elo, ei, agg_lo,
                   sib, ebuf, zbuf, spacc, sem_l, sem_g, sem_s)

    @pl.when(c == 1)
    def _():
        _edge_half(s, hhi, ehi, ei, agg_hi,
                   sib, ebuf, zbuf, spacc, sem_l, sem_g, sem_s)


def _edge_call(hlo, hhi, elo, ehi, ei_flat):
    mesh = plsc.VectorSubcoreMesh(core_axis_name="c", subcore_axis_name="s")
    fn = pl.kernel(
        _edge_body,
        out_type=(
            jax.ShapeDtypeStruct((N_PAD, HH), f32),
            jax.ShapeDtypeStruct((N_PAD, HH), f32),
        ),
        mesh=mesh,
        scratch_types=[
            pltpu.VMEM((NBUF, 2, K), jnp.int32),
            pltpu.VMEM((NBUF, K, HH), f32),
            pltpu.VMEM((ZROWS, HH), f32),
            pltpu.VMEM_SHARED((N_PAD, HH), f32),
            pltpu.SemaphoreType.DMA,
            pltpu.SemaphoreType.DMA,
            pltpu.SemaphoreType.DMA,
        ],
        compiler_params=pltpu.CompilerParams(use_tc_tiling_on_sc=False),
    )
    return fn(hlo, hhi, elo, ehi, ei_flat)


def kernel(feat, eweight, edge_index, Wn, We, params, Wo, bo):
    Wn_p = jnp.pad(Wn, ((0, H - Wn.shape[0]), (0, 0)))
    wn_lo = _kron16(Wn_p[:, :HH])
    wn_hi = _kron16(Wn_p[:, HH:])
    eye32 = jnp.eye(32, dtype=f32)
    we_lo = jnp.kron(eye32, We[:, :HH])
    we_hi = jnp.kron(eye32, We[:, HH:])
    fold = jnp.kron(jnp.ones((PK, 1), f32), jnp.eye(HH, dtype=f32))
    wo2 = Wo.reshape(2, HH)
    bo_r = bo.reshape(1, 1)

    featp = jnp.pad(feat, ((0, N_PAD - N), (0, H - feat.shape[1])))
    featp = featp.reshape(NP, PK * H)
    ewp = eweight.reshape(E * 6 // 192, 192)
    ei_flat = edge_index.reshape(2 * E)

    h_lo, h_hi = _embed(featp, wn_lo, wn_hi, BPN)
    e_lo, e_hi = _embed(ewp, we_lo, we_hi, BPE)
    e_lo2 = e_lo.reshape(E, HH)
    e_hi2 = e_hi.reshape(E, HH)

    out = None
    for li, (W1, b1, W2, b2) in enumerate(params):
        kws = (_kron16(W1[:HH, :HH]), _kron16(W1[:HH, HH:]),
               _kron16(W1[HH:, :HH]), _kron16(W1[HH:, HH:]),
               _kron16(W2[:HH, :HH]), _kron16(W2[:HH, HH:]),
               _kron16(W2[HH:, :HH]), _kron16(W2[HH:, HH:]))
        bb = jnp.stack([jnp.tile(b1[:HH], PK), jnp.tile(b1[HH:], PK),
                        jnp.tile(b2[:HH], PK), jnp.tile(b2[HH:], PK)])
        agg_lo, agg_hi = _edge_call(h_lo.reshape(N_PAD, HH),
                                    h_hi.reshape(N_PAD, HH),
                                    e_lo2, e_hi2, ei_flat)
        al = agg_lo.reshape(N_PAD * HH)
        ah = agg_hi.reshape(N_PAD * HH)
        if li < 3:
            h_lo, h_hi = _mlp(h_lo, h_hi, al, ah, kws, bb)
        else:
            out = _mlp_readout(h_lo, h_hi, al, ah, kws, bb, fold, wo2, bo_r)
    return out
